# baseline (speedup 1.0000x reference)
import functools

import jax
import jax.numpy as jnp
from jax import lax
from jax.experimental import pallas as pl
from jax.experimental.pallas import tpu as pltpu
from jax.experimental.pallas import tpu_sc as plsc

N_NODES = 100000
N_ELEMENTS = 200000
D_FEAT = 128

NC, NS, L = 2, 16, 16
NW = NC * NS
K = 128
NCH = 50
EW = NCH * K
E_PAD = NW * EW
SLICE = 6256
N_PAD = NS * SLICE

_mesh = plsc.VectorSubcoreMesh(
    core_axis_name="c", subcore_axis_name="s", num_cores=NC, num_subcores=NS
)


@functools.partial(
    pl.kernel,
    out_type=(
        jax.ShapeDtypeStruct((N_PAD,), jnp.float32),
        jax.ShapeDtypeStruct((N_PAD,), jnp.float32),
    ),
    mesh=_mesh,
    scratch_types=[
        pltpu.VMEM((EW,), jnp.int32),
        pltpu.VMEM((EW,), jnp.int32),
        pltpu.VMEM((EW,), jnp.int32),
        pltpu.VMEM((EW,), jnp.float32),
        pltpu.VMEM((EW,), jnp.float32),
        pltpu.VMEM((EW,), jnp.float32),
        pltpu.VMEM((EW,), jnp.float32),
        pltpu.VMEM((EW,), jnp.float32),
        pltpu.VMEM((EW,), jnp.float32),
        pltpu.VMEM((EW,), jnp.float32),
        pltpu.VMEM((SLICE,), jnp.float32),
        pltpu.VMEM_SHARED((N_PAD,), jnp.float32),
        pltpu.SemaphoreType.DMA,
    ],
)
def _sc_node_weights(xs_hbm, ys_hbm, i0_hbm, i1_hbm, i2_hbm,
                     out0_hbm, out1_hbm,
                     i0_v, i1_v, i2_v, x0_v, x1_v, x2_v, y0_v, y1_v, y2_v,
                     w_v, z_v, s_sh, sem):
    cid = lax.axis_index("c")
    sid = lax.axis_index("s")
    wid = sid * NC + cid

    def _zero(j, carry):
        z_v[pl.ds(j * L, L)] = jnp.zeros((L,), jnp.float32)
        return carry

    lax.fori_loop(0, SLICE // L, _zero, 0)
    pltpu.sync_copy(z_v, s_sh.at[pl.ds(sid * SLICE, SLICE)])

    c0 = pltpu.async_copy(i0_hbm.at[wid], i0_v, sem)
    c1 = pltpu.async_copy(i1_hbm.at[wid], i1_v, sem)
    c2 = pltpu.async_copy(i2_hbm.at[wid], i2_v, sem)
    c0.wait()
    c1.wait()
    c2.wait()

    plsc.subcore_barrier()

    g0 = pltpu.async_copy(xs_hbm.at[i0_v], x0_v, sem)
    g1 = pltpu.async_copy(xs_hbm.at[i1_v], x1_v, sem)
    g2 = pltpu.async_copy(xs_hbm.at[i2_v], x2_v, sem)
    g3 = pltpu.async_copy(ys_hbm.at[i0_v], y0_v, sem)
    g4 = pltpu.async_copy(ys_hbm.at[i1_v], y1_v, sem)
    g5 = pltpu.async_copy(ys_hbm.at[i2_v], y2_v, sem)
    g0.wait(); g1.wait(); g2.wait(); g3.wait(); g4.wait(); g5.wait()

    def _wcompute(t, carry):
        sl = pl.ds(t * L, L)
        ax = x1_v[sl] - x0_v[sl]
        ay = y1_v[sl] - y0_v[sl]
        bx = x2_v[sl] - x0_v[sl]
        by = y2_v[sl] - y0_v[sl]
        det = ax * by - bx * ay
        w_v[sl] = jnp.abs(det) * jnp.float32(1.0 / 6.0)
        return carry

    lax.fori_loop(0, EW // L, _wcompute, 0)

    s0c = pltpu.async_copy(w_v, s_sh.at[i0_v], sem, add=True)
    s1c = pltpu.async_copy(w_v, s_sh.at[i1_v], sem, add=True)
    s2c = pltpu.async_copy(w_v, s_sh.at[i2_v], sem, add=True)
    s0c.wait(); s1c.wait(); s2c.wait()

    plsc.subcore_barrier()
    sl_out = pl.ds(sid * SLICE, SLICE)
    pltpu.sync_copy(s_sh.at[sl_out], z_v)

    @pl.when(cid == 0)
    def _():
        pltpu.sync_copy(z_v, out0_hbm.at[sl_out])

    @pl.when(cid == 1)
    def _():
        pltpu.sync_copy(z_v, out1_hbm.at[sl_out])


_BN = 4000


def _tc_body(s0_ref, s1_ref, v_ref, o_ref):
    i = pl.program_id(0)

    @pl.when(i == 0)
    def _():
        o_ref[...] = jnp.zeros_like(o_ref)

    s = s0_ref[...] + s1_ref[...]
    o_ref[...] += jnp.sum(v_ref[...] * s, axis=0, keepdims=True)


def kernel(coords, elements, nodal_values):
    el = elements.astype(jnp.int32)
    xs = jnp.zeros((N_PAD,), jnp.float32).at[:N_NODES].set(coords[:, 0])
    ys = jnp.zeros((N_PAD,), jnp.float32).at[:N_NODES].set(coords[:, 1])
    idx = jnp.full((3, E_PAD), N_NODES, jnp.int32).at[:, :N_ELEMENTS].set(el.T)
    idx = idx.reshape(3, NW, EW)

    sa, sb = _sc_node_weights(xs, ys, idx[0], idx[1], idx[2])

    s0 = sa.reshape(N_PAD, 1)
    s1 = sb.reshape(N_PAD, 1)
    out = pl.pallas_call(
        _tc_body,
        grid=(N_NODES // _BN,),
        in_specs=[
            pl.BlockSpec((_BN, 1), lambda i: (i, 0)),
            pl.BlockSpec((_BN, 1), lambda i: (i, 0)),
            pl.BlockSpec((_BN, D_FEAT), lambda i: (i, 0)),
        ],
        out_specs=pl.BlockSpec((1, D_FEAT), lambda i: (0, 0)),
        out_shape=jax.ShapeDtypeStruct((1, D_FEAT), jnp.float32),
    )(s0, s1, nodal_values)
    return out[0]


# speedup vs baseline: 1.1538x; 1.1538x over previous
import functools

import jax
import jax.numpy as jnp
from jax import lax
from jax.experimental import pallas as pl
from jax.experimental.pallas import tpu as pltpu
from jax.experimental.pallas import tpu_sc as plsc

N_NODES = 100000
N_ELEMENTS = 200000
D_FEAT = 128

NC, NS, L = 2, 16, 16
NW = NC * NS
K = 1600
NCH = 4
EW = NCH * K
E_PAD = NW * EW
SLICE = 6256
N_PAD = NS * SLICE

_mesh = plsc.VectorSubcoreMesh(
    core_axis_name="c", subcore_axis_name="s", num_cores=NC, num_subcores=NS
)

_scratch = (
    [pltpu.VMEM((K,), jnp.int32) for _ in range(3 * NCH)]
    + [pltpu.VMEM((K,), jnp.float32) for _ in range(12)]
    + [pltpu.VMEM((K,), jnp.float32) for _ in range(2)]
    + [
        pltpu.VMEM((SLICE,), jnp.float32),
        pltpu.VMEM_SHARED((N_PAD,), jnp.float32),
        pltpu.SemaphoreType.DMA,
        pltpu.SemaphoreType.DMA,
        pltpu.SemaphoreType.DMA,
        pltpu.SemaphoreType.DMA,
    ]
)


@functools.partial(
    pl.kernel,
    out_type=(
        jax.ShapeDtypeStruct((N_PAD,), jnp.float32),
        jax.ShapeDtypeStruct((N_PAD,), jnp.float32),
    ),
    mesh=_mesh,
    scratch_types=_scratch,
)
def _sc_node_weights(xs_hbm, ys_hbm, idx_hbm, out0_hbm, out1_hbm, *scr):
    iv = [scr[0:NCH], scr[NCH:2 * NCH], scr[2 * NCH:3 * NCH]]
    xv = [scr[12 + 2 * j: 14 + 2 * j] for j in range(3)]
    yv = [scr[18 + 2 * j: 20 + 2 * j] for j in range(3)]
    wv = scr[24:26]
    z_v = scr[26]
    s_sh = scr[27]
    semg = scr[28:30]
    sems = scr[30:32]

    cid = lax.axis_index("c")
    sid = lax.axis_index("s")
    wid = sid * NC + cid

    def _zero(j, carry):
        z_v[pl.ds(j * L, L)] = jnp.zeros((L,), jnp.float32)
        return carry

    lax.fori_loop(0, SLICE // L, _zero, 0)
    pltpu.sync_copy(z_v, s_sh.at[pl.ds(sid * SLICE, SLICE)])

    plsc.subcore_barrier()

    stage = [
        pltpu.async_copy(idx_hbm.at[j, wid, c], iv[j][c], semg[0])
        for j in range(3)
        for c in range(NCH)
    ]
    for h in stage:
        h.wait()

    def fire_gathers(c):
        p = c % 2
        hs = []
        for j in range(3):
            hs.append(
                pltpu.async_copy(xs_hbm.at[iv[j][c]], xv[j][p], semg[p]))
            hs.append(
                pltpu.async_copy(ys_hbm.at[iv[j][c]], yv[j][p], semg[p]))
        return hs

    def fire_scatters(c):
        p = c % 2
        return [
            pltpu.async_copy(wv[p], s_sh.at[iv[j][c]], sems[p], add=True)
            for j in range(3)
        ]

    def compute(c):
        p = c % 2

        def _wcompute(t, carry):
            sl = pl.ds(t * L, L)
            ax = xv[1][p][sl] - xv[0][p][sl]
            ay = yv[1][p][sl] - yv[0][p][sl]
            bx = xv[2][p][sl] - xv[0][p][sl]
            by = yv[2][p][sl] - yv[0][p][sl]
            det = ax * by - bx * ay
            wv[p][sl] = jnp.abs(det) * jnp.float32(1.0 / 6.0)
            return carry

        lax.fori_loop(0, K // L, _wcompute, 0)

    handles_g = [None] * NCH
    handles_s = [None] * NCH
    handles_g[0] = fire_gathers(0)
    for c in range(NCH):
        if c + 1 < NCH:
            handles_g[c + 1] = fire_gathers(c + 1)
        for h in handles_g[c]:
            h.wait()
        if c >= 2:
            for h in handles_s[c - 2]:
                h.wait()
        compute(c)
        handles_s[c] = fire_scatters(c)
    for h in handles_s[NCH - 2]:
        h.wait()
    for h in handles_s[NCH - 1]:
        h.wait()

    plsc.subcore_barrier()
    sl_out = pl.ds(sid * SLICE, SLICE)
    pltpu.sync_copy(s_sh.at[sl_out], z_v)

    @pl.when(cid == 0)
    def _():
        pltpu.sync_copy(z_v, out0_hbm.at[sl_out])

    @pl.when(cid == 1)
    def _():
        pltpu.sync_copy(z_v, out1_hbm.at[sl_out])


_BN = 4000


def _tc_body(s0_ref, s1_ref, v_ref, o_ref):
    i = pl.program_id(0)

    @pl.when(i == 0)
    def _():
        o_ref[...] = jnp.zeros_like(o_ref)

    s = s0_ref[...] + s1_ref[...]
    o_ref[...] += jnp.sum(v_ref[...] * s, axis=0, keepdims=True)


def kernel(coords, elements, nodal_values):
    el = elements.astype(jnp.int32)
    xs = jnp.zeros((N_PAD,), jnp.float32).at[:N_NODES].set(coords[:, 0])
    ys = jnp.zeros((N_PAD,), jnp.float32).at[:N_NODES].set(coords[:, 1])
    idx = jnp.full((3, E_PAD), N_NODES, jnp.int32).at[:, :N_ELEMENTS].set(el.T)
    idx = idx.reshape(3, NW, NCH, K)

    sa, sb = _sc_node_weights(xs, ys, idx)

    s0 = sa.reshape(N_PAD, 1)
    s1 = sb.reshape(N_PAD, 1)
    out = pl.pallas_call(
        _tc_body,
        grid=(N_NODES // _BN,),
        in_specs=[
            pl.BlockSpec((_BN, 1), lambda i: (i, 0)),
            pl.BlockSpec((_BN, 1), lambda i: (i, 0)),
            pl.BlockSpec((_BN, D_FEAT), lambda i: (i, 0)),
        ],
        out_specs=pl.BlockSpec((1, D_FEAT), lambda i: (0, 0)),
        out_shape=jax.ShapeDtypeStruct((1, D_FEAT), jnp.float32),
    )(s0, s1, nodal_values)
    return out[0]


# speedup vs baseline: 1.1654x; 1.0101x over previous
import functools

import jax
import jax.numpy as jnp
from jax import lax
from jax.experimental import pallas as pl
from jax.experimental.pallas import tpu as pltpu
from jax.experimental.pallas import tpu_sc as plsc

N_NODES = 100000
N_ELEMENTS = 200000
D_FEAT = 128

NC, NS, L = 2, 16, 16
NW = NC * NS
K = 1600
NCH = 4
EW = NCH * K
E_PAD = NW * EW
SLICE = 6256
N_PAD = NS * SLICE

_mesh = plsc.VectorSubcoreMesh(
    core_axis_name="c", subcore_axis_name="s", num_cores=NC, num_subcores=NS
)

_scratch = (
    [pltpu.VMEM((K,), jnp.int32) for _ in range(3 * NCH)]
    + [pltpu.VMEM((K,), jnp.float32) for _ in range(12)]
    + [pltpu.VMEM((K,), jnp.float32) for _ in range(2)]
    + [
        pltpu.VMEM((SLICE,), jnp.float32),
        pltpu.VMEM_SHARED((N_PAD,), jnp.float32),
        pltpu.SemaphoreType.DMA,
        pltpu.SemaphoreType.DMA,
        pltpu.SemaphoreType.DMA,
        pltpu.SemaphoreType.DMA,
    ]
)


@functools.partial(
    pl.kernel,
    out_type=(
        jax.ShapeDtypeStruct((N_PAD,), jnp.float32),
        jax.ShapeDtypeStruct((N_PAD,), jnp.float32),
    ),
    mesh=_mesh,
    scratch_types=_scratch,
)
def _sc_node_weights(xs_hbm, ys_hbm, idx_hbm, out0_hbm, out1_hbm, *scr):
    iv = [scr[0:NCH], scr[NCH:2 * NCH], scr[2 * NCH:3 * NCH]]
    xv = [scr[12 + 2 * j: 14 + 2 * j] for j in range(3)]
    yv = [scr[18 + 2 * j: 20 + 2 * j] for j in range(3)]
    wv = scr[24:26]
    z_v = scr[26]
    s_sh = scr[27]
    semg = scr[28:30]
    sems = scr[30:32]

    cid = lax.axis_index("c")
    sid = lax.axis_index("s")
    wid = sid * NC + cid

    def _zero(j, carry):
        z_v[pl.ds(j * L, L)] = jnp.zeros((L,), jnp.float32)
        return carry

    lax.fori_loop(0, SLICE // L, _zero, 0)
    pltpu.sync_copy(z_v, s_sh.at[pl.ds(sid * SLICE, SLICE)])

    plsc.subcore_barrier()

    stage = [
        pltpu.async_copy(idx_hbm.at[j, wid, c], iv[j][c], semg[0])
        for j in range(3)
        for c in range(NCH)
    ]
    for h in stage:
        h.wait()

    def fire_gathers(c):
        p = c % 2
        hs = []
        for j in range(3):
            hs.append(
                pltpu.async_copy(xs_hbm.at[iv[j][c]], xv[j][p], semg[p]))
            hs.append(
                pltpu.async_copy(ys_hbm.at[iv[j][c]], yv[j][p], semg[p]))
        return hs

    def fire_scatters(c):
        p = c % 2
        return [
            pltpu.async_copy(wv[p], s_sh.at[iv[j][c]], sems[p], add=True)
            for j in range(3)
        ]

    def compute(c):
        p = c % 2

        def _wcompute(t, carry):
            sl = pl.ds(t * L, L)
            ax = xv[1][p][sl] - xv[0][p][sl]
            ay = yv[1][p][sl] - yv[0][p][sl]
            bx = xv[2][p][sl] - xv[0][p][sl]
            by = yv[2][p][sl] - yv[0][p][sl]
            det = ax * by - bx * ay
            wv[p][sl] = jnp.abs(det) * jnp.float32(1.0 / 6.0)
            return carry

        lax.fori_loop(0, K // L, _wcompute, 0)

    handles_g = [None] * NCH
    handles_s = [None] * NCH
    handles_g[0] = fire_gathers(0)
    for c in range(NCH):
        if c + 1 < NCH:
            handles_g[c + 1] = fire_gathers(c + 1)
        for h in handles_g[c]:
            h.wait()
        if c >= 2:
            for h in handles_s[c - 2]:
                h.wait()
        compute(c)
        handles_s[c] = fire_scatters(c)
    for h in handles_s[NCH - 2]:
        h.wait()
    for h in handles_s[NCH - 1]:
        h.wait()

    plsc.subcore_barrier()
    sl_out = pl.ds(sid * SLICE, SLICE)
    pltpu.sync_copy(s_sh.at[sl_out], z_v)

    @pl.when(cid == 0)
    def _():
        pltpu.sync_copy(z_v, out0_hbm.at[sl_out])

    @pl.when(cid == 1)
    def _():
        pltpu.sync_copy(z_v, out1_hbm.at[sl_out])


_BN = 4000


def _tc_body(s0_ref, s1_ref, v_ref, o_ref):
    i = pl.program_id(0)

    @pl.when(i == 0)
    def _():
        o_ref[...] = jnp.zeros_like(o_ref)

    s = s0_ref[...] + s1_ref[...]
    o_ref[...] += jnp.sum(v_ref[...] * s, axis=0, keepdims=True)


def kernel(coords, elements, nodal_values):
    el = elements.astype(jnp.int32)
    xs = coords[:, 0]
    ys = coords[:, 1]
    idx = jnp.zeros((3, E_PAD), jnp.int32).at[:, :N_ELEMENTS].set(el.T)
    idx = idx.reshape(3, NW, NCH, K)

    sa, sb = _sc_node_weights(xs, ys, idx)

    s0 = sa.reshape(N_PAD, 1)
    s1 = sb.reshape(N_PAD, 1)
    out = pl.pallas_call(
        _tc_body,
        grid=(N_NODES // _BN,),
        in_specs=[
            pl.BlockSpec((_BN, 1), lambda i: (i, 0)),
            pl.BlockSpec((_BN, 1), lambda i: (i, 0)),
            pl.BlockSpec((_BN, D_FEAT), lambda i: (i, 0)),
        ],
        out_specs=pl.BlockSpec((1, D_FEAT), lambda i: (0, 0)),
        out_shape=jax.ShapeDtypeStruct((1, D_FEAT), jnp.float32),
    )(s0, s1, nodal_values)
    return out[0]


# speedup vs baseline: 1.2807x; 1.0989x over previous
import functools

import jax
import jax.numpy as jnp
from jax import lax
from jax.experimental import pallas as pl
from jax.experimental.pallas import tpu as pltpu
from jax.experimental.pallas import tpu_sc as plsc

N_NODES = 100000
N_ELEMENTS = 200000
D_FEAT = 128

NC, NS, L = 2, 16, 16
NW = NC * NS
K = 800
NCH = 8
EW = NCH * K
E_PAD = NW * EW
SLICE = 6256
N_PAD = NS * SLICE

_mesh = plsc.VectorSubcoreMesh(
    core_axis_name="c", subcore_axis_name="s", num_cores=NC, num_subcores=NS
)

_scratch = (
    [pltpu.VMEM((K,), jnp.int32) for _ in range(3 * NCH)]
    + [pltpu.VMEM((K,), jnp.float32) for _ in range(12)]
    + [pltpu.VMEM((K,), jnp.float32) for _ in range(2)]
    + [
        pltpu.VMEM((SLICE,), jnp.float32),
        pltpu.VMEM_SHARED((N_PAD,), jnp.float32),
        pltpu.SemaphoreType.DMA,
        pltpu.SemaphoreType.DMA,
        pltpu.SemaphoreType.DMA,
        pltpu.SemaphoreType.DMA,
    ]
)


@functools.partial(
    pl.kernel,
    out_type=(
        jax.ShapeDtypeStruct((N_PAD,), jnp.float32),
        jax.ShapeDtypeStruct((N_PAD,), jnp.float32),
    ),
    mesh=_mesh,
    scratch_types=_scratch,
)
def _sc_node_weights(xs_hbm, ys_hbm, idx_hbm, out0_hbm, out1_hbm, *scr):
    iv = [scr[0:NCH], scr[NCH:2 * NCH], scr[2 * NCH:3 * NCH]]
    b = 3 * NCH
    xv = [scr[b + 2 * j: b + 2 * j + 2] for j in range(3)]
    yv = [scr[b + 6 + 2 * j: b + 8 + 2 * j] for j in range(3)]
    wv = scr[b + 12: b + 14]
    z_v = scr[b + 14]
    s_sh = scr[b + 15]
    semg = scr[b + 16: b + 18]
    sems = scr[b + 18: b + 20]

    cid = lax.axis_index("c")
    sid = lax.axis_index("s")
    wid = sid * NC + cid

    def _zero(j, carry):
        z_v[pl.ds(j * L, L)] = jnp.zeros((L,), jnp.float32)
        return carry

    lax.fori_loop(0, SLICE // L, _zero, 0)
    pltpu.sync_copy(z_v, s_sh.at[pl.ds(sid * SLICE, SLICE)])

    plsc.subcore_barrier()

    stage = [
        pltpu.async_copy(idx_hbm.at[j, wid, c], iv[j][c], semg[0])
        for j in range(3)
        for c in range(NCH)
    ]
    for h in stage:
        h.wait()

    def fire_gathers(c):
        p = c % 2
        hs = []
        for j in range(3):
            hs.append(
                pltpu.async_copy(xs_hbm.at[iv[j][c]], xv[j][p], semg[p]))
            hs.append(
                pltpu.async_copy(ys_hbm.at[iv[j][c]], yv[j][p], semg[p]))
        return hs

    def fire_scatters(c):
        p = c % 2
        return [
            pltpu.async_copy(wv[p], s_sh.at[iv[j][c]], sems[p], add=True)
            for j in range(3)
        ]

    def compute(c):
        p = c % 2

        def _wcompute(t, carry):
            sl = pl.ds(t * L, L)
            ax = xv[1][p][sl] - xv[0][p][sl]
            ay = yv[1][p][sl] - yv[0][p][sl]
            bx = xv[2][p][sl] - xv[0][p][sl]
            by = yv[2][p][sl] - yv[0][p][sl]
            det = ax * by - bx * ay
            wv[p][sl] = jnp.abs(det) * jnp.float32(1.0 / 6.0)
            return carry

        lax.fori_loop(0, K // L, _wcompute, 0)

    handles_g = [None] * NCH
    handles_s = [None] * NCH
    handles_g[0] = fire_gathers(0)
    for c in range(NCH):
        if c + 1 < NCH:
            handles_g[c + 1] = fire_gathers(c + 1)
        for h in handles_g[c]:
            h.wait()
        if c >= 2:
            for h in handles_s[c - 2]:
                h.wait()
        compute(c)
        handles_s[c] = fire_scatters(c)
    for h in handles_s[NCH - 2]:
        h.wait()
    for h in handles_s[NCH - 1]:
        h.wait()

    plsc.subcore_barrier()
    sl_out = pl.ds(sid * SLICE, SLICE)
    pltpu.sync_copy(s_sh.at[sl_out], z_v)

    @pl.when(cid == 0)
    def _():
        pltpu.sync_copy(z_v, out0_hbm.at[sl_out])

    @pl.when(cid == 1)
    def _():
        pltpu.sync_copy(z_v, out1_hbm.at[sl_out])


_BN = 4000


def _tc_body(s0_ref, s1_ref, v_ref, o_ref):
    i = pl.program_id(0)

    @pl.when(i == 0)
    def _():
        o_ref[...] = jnp.zeros_like(o_ref)

    s = s0_ref[...] + s1_ref[...]
    o_ref[...] += jnp.sum(v_ref[...] * s, axis=0, keepdims=True)


def kernel(coords, elements, nodal_values):
    el = elements.astype(jnp.int32)
    xs = coords[:, 0]
    ys = coords[:, 1]
    idx = jnp.zeros((3, E_PAD), jnp.int32).at[:, :N_ELEMENTS].set(el.T)
    idx = idx.reshape(3, NW, NCH, K)

    sa, sb = _sc_node_weights(xs, ys, idx)

    s0 = sa.reshape(N_PAD, 1)
    s1 = sb.reshape(N_PAD, 1)
    out = pl.pallas_call(
        _tc_body,
        grid=(N_NODES // _BN,),
        in_specs=[
            pl.BlockSpec((_BN, 1), lambda i: (i, 0)),
            pl.BlockSpec((_BN, 1), lambda i: (i, 0)),
            pl.BlockSpec((_BN, D_FEAT), lambda i: (i, 0)),
        ],
        out_specs=pl.BlockSpec((1, D_FEAT), lambda i: (0, 0)),
        out_shape=jax.ShapeDtypeStruct((1, D_FEAT), jnp.float32),
    )(s0, s1, nodal_values)
    return out[0]
